# baseline (speedup 1.0000x reference)
import functools

import jax
import jax.numpy as jnp
from jax import lax
from jax.experimental import pallas as pl
from jax.experimental.pallas import tpu as pltpu
from jax.experimental.pallas import tpu_sc as plsc

B = 10000
NUM_RELS = 1315
FEAT_DIM = 256
REL_DIM = 32
OUT_DIM = 32
BLK = 512
GRID = -(-B // BLK)
BPAD = GRID * BLK
GD = 128


def _sc_gather(table, eid_pad):
    info = plsc.get_sparse_core_info()
    nw = info.num_cores * info.num_subcores
    b_per_w = BPAD // nw
    mesh = plsc.VectorSubcoreMesh(core_axis_name="c", subcore_axis_name="s")

    @functools.partial(
        pl.kernel, mesh=mesh,
        out_type=jax.ShapeDtypeStruct((BPAD, GD), jnp.float32),
        scratch_types=[pltpu.VMEM((b_per_w,), jnp.int32),
                       pltpu.VMEM((b_per_w, GD), jnp.float32),
                       pltpu.SemaphoreType.DMA],
    )
    def k(table_hbm, idx_hbm, out_hbm, idx_v, rows_v, sem):
        wid = lax.axis_index("s") * info.num_cores + lax.axis_index("c")
        base = wid * b_per_w
        pltpu.sync_copy(idx_hbm.at[pl.ds(base, b_per_w)], idx_v)
        pltpu.async_copy(table_hbm.at[idx_v], rows_v, sem).wait()
        pltpu.sync_copy(rows_v, out_hbm.at[pl.ds(base, b_per_w)])

    return k(table, eid_pad)


def _main_kernel(hi_ref, ho_ref, hf_ref, pi_ref, po_ref, pf_ref,
                 ni_ref, no_ref, nf_ref, rel_ref,
                 inW_ref, inb_ref, outW_ref, outb_ref,
                 entW_ref, entb_ref,
                 dp_ref, dn_ref,
                 mtin_ref, mtout_ref, bias_ref):
    f32 = jnp.float32

    @pl.when(pl.program_id(0) == 0)
    def _prep():
        E1 = entW_ref[FEAT_DIM:FEAT_DIM + REL_DIM, :]
        E2 = entW_ref[FEAT_DIM + REL_DIM:, :]
        C_in = jnp.dot(inW_ref[...], E1, preferred_element_type=f32)
        C_out = jnp.dot(outW_ref[...], E2, preferred_element_type=f32)
        mtin_ref[...] = jnp.dot(rel_ref[...], C_in,
                                preferred_element_type=f32).T
        mtout_ref[...] = jnp.dot(rel_ref[...], C_out,
                                 preferred_element_type=f32).T
        bias_ref[...] = (jnp.dot(inb_ref[...], E1, preferred_element_type=f32)
                         + jnp.dot(outb_ref[...], E2,
                                   preferred_element_type=f32)
                         + entb_ref[...])

    E0 = entW_ref[:FEAT_DIM, :]
    biasT = bias_ref[...].T

    def embed(incT_in, incT_out, nf):
        f = jnp.dot(nf, E0, preferred_element_type=f32).T
        z = (f
             + jnp.dot(mtin_ref[...], incT_in, preferred_element_type=f32)
             + jnp.dot(mtout_ref[...], incT_out, preferred_element_type=f32)
             + biasT)
        n = jnp.sqrt(jnp.sum(z * z, axis=0, keepdims=True))
        return z / jnp.maximum(n, 1e-12)

    h = embed(hi_ref[...], ho_ref[...], hf_ref[...])
    p = embed(pi_ref[...], po_ref[...], pf_ref[...])
    t = embed(ni_ref[...], no_ref[...], nf_ref[...])

    dp_ref[...] = h - p
    dn_ref[...] = h - t


def _score_kernel(dp_ref, dn_ref, g_ref, rTW_ref, rTb_ref,
                  pos_ref, neg_ref):
    f32 = jnp.float32
    r = (jnp.dot(g_ref[...], rTW_ref[...], preferred_element_type=f32)
         + rTb_ref[...]).T
    rn = jnp.sqrt(jnp.sum(r * r, axis=0, keepdims=True))
    r = r / jnp.maximum(rn, 1e-12)
    dp = dp_ref[...] + r
    dn = dn_ref[...] + r
    pos_ref[0, 0, :] = jnp.sqrt(jnp.sum(dp * dp, axis=0))
    neg_ref[0, 0, :] = jnp.sqrt(jnp.sum(dn * dn, axis=0))


def kernel(h_in_inc, h_out_inc, h_node_feat, eid, pos_t_in_inc, pos_t_out_inc,
           pos_t_node_feat, neg_t_in_inc, neg_t_out_inc, neg_t_node_feat,
           rel_emb, in_W, in_b, out_W, out_b, ent_W, ent_b, relT_W, relT_b):
    eid_pad = jnp.pad(eid.astype(jnp.int32), (0, BPAD - B))
    table = jnp.pad(rel_emb, ((0, 0), (0, GD - REL_DIM)))
    g = _sc_gather(table, eid_pad)
    rTW_pad = jnp.pad(relT_W, ((0, GD - REL_DIM), (0, 0)))

    incT_spec = pl.BlockSpec((NUM_RELS, BLK), lambda i: (0, i))
    feat_spec = pl.BlockSpec((BLK, FEAT_DIM), lambda i: (i, 0))
    d_spec = pl.BlockSpec((REL_DIM, BLK), lambda i: (0, i))

    def full(shape):
        return pl.BlockSpec(shape, lambda i: (0,) * len(shape))

    dp, dn = pl.pallas_call(
        _main_kernel,
        grid=(GRID,),
        in_specs=[incT_spec, incT_spec, feat_spec,
                  incT_spec, incT_spec, feat_spec,
                  incT_spec, incT_spec, feat_spec,
                  full((NUM_RELS, REL_DIM)),
                  full((REL_DIM, REL_DIM)), full((1, REL_DIM)),
                  full((REL_DIM, REL_DIM)), full((1, REL_DIM)),
                  full((FEAT_DIM + 2 * REL_DIM, OUT_DIM)), full((1, OUT_DIM))],
        out_specs=[d_spec, d_spec],
        out_shape=[jax.ShapeDtypeStruct((REL_DIM, BPAD), jnp.float32),
                   jax.ShapeDtypeStruct((REL_DIM, BPAD), jnp.float32)],
        scratch_shapes=[pltpu.VMEM((REL_DIM, NUM_RELS), jnp.float32),
                        pltpu.VMEM((REL_DIM, NUM_RELS), jnp.float32),
                        pltpu.VMEM((1, OUT_DIM), jnp.float32)],
    )(h_in_inc.T, h_out_inc.T, h_node_feat,
      pos_t_in_inc.T, pos_t_out_inc.T, pos_t_node_feat,
      neg_t_in_inc.T, neg_t_out_inc.T, neg_t_node_feat,
      rel_emb,
      in_W, in_b.reshape(1, REL_DIM), out_W, out_b.reshape(1, REL_DIM),
      ent_W, ent_b.reshape(1, OUT_DIM))

    g_spec = pl.BlockSpec((BLK, GD), lambda i: (i, 0))
    out_spec = pl.BlockSpec((1, 1, BLK), lambda i: (i, 0, 0))
    pos, neg = pl.pallas_call(
        _score_kernel,
        grid=(GRID,),
        in_specs=[d_spec, d_spec, g_spec,
                  full((GD, OUT_DIM)), full((1, OUT_DIM))],
        out_specs=[out_spec, out_spec],
        out_shape=[jax.ShapeDtypeStruct((GRID, 1, BLK), jnp.float32),
                   jax.ShapeDtypeStruct((GRID, 1, BLK), jnp.float32)],
    )(dp, dn, g, rTW_pad, relT_b.reshape(1, OUT_DIM))
    return pos.reshape(BPAD)[:B], neg.reshape(BPAD)[:B]


# speedup vs baseline: 1.3760x; 1.3760x over previous
import jax
import jax.numpy as jnp
from jax.experimental import pallas as pl
from jax.experimental.pallas import tpu as pltpu

B = 10000
NUM_RELS = 1315
FEAT_DIM = 256
REL_DIM = 32
OUT_DIM = 32
BLK = 640
GRID = -(-B // BLK)
BPAD = GRID * BLK


def _fused_kernel(hi_ref, ho_ref, hf_ref, pi_ref, po_ref, pf_ref,
                  ni_ref, no_ref, nf_ref, eid_ref, rel_ref,
                  inW_ref, inb_ref, outW_ref, outb_ref,
                  entW_ref, entb_ref, rTW_ref, rTb_ref,
                  pos_ref, neg_ref,
                  mtin_ref, mtout_ref, tt_ref, bias_ref):
    f32 = jnp.float32

    @pl.when(pl.program_id(0) == 0)
    def _prep():
        E1 = entW_ref[FEAT_DIM:FEAT_DIM + REL_DIM, :]
        E2 = entW_ref[FEAT_DIM + REL_DIM:, :]
        C_in = jnp.dot(inW_ref[...], E1, preferred_element_type=f32)
        C_out = jnp.dot(outW_ref[...], E2, preferred_element_type=f32)
        mtin_ref[...] = jnp.dot(rel_ref[...], C_in,
                                preferred_element_type=f32).T
        mtout_ref[...] = jnp.dot(rel_ref[...], C_out,
                                 preferred_element_type=f32).T
        tt_ref[...] = jnp.dot(rel_ref[...], rTW_ref[...],
                              preferred_element_type=f32).T
        bias_ref[...] = (jnp.dot(inb_ref[...], E1, preferred_element_type=f32)
                         + jnp.dot(outb_ref[...], E2,
                                   preferred_element_type=f32)
                         + entb_ref[...])

    E0 = entW_ref[:FEAT_DIM, :]
    biasT = bias_ref[...].T

    def embed(incT_in, incT_out, nf):
        f = jnp.dot(nf, E0, preferred_element_type=f32).T
        z = (f
             + jnp.dot(mtin_ref[...], incT_in, preferred_element_type=f32)
             + jnp.dot(mtout_ref[...], incT_out, preferred_element_type=f32)
             + biasT)
        n = jnp.sqrt(jnp.sum(z * z, axis=0, keepdims=True))
        return z / jnp.maximum(n, 1e-12)

    h = embed(hi_ref[...], ho_ref[...], hf_ref[...])
    p = embed(pi_ref[...], po_ref[...], pf_ref[...])
    t = embed(ni_ref[...], no_ref[...], nf_ref[...])

    e = eid_ref[0, :]
    onehotT = (jax.lax.broadcasted_iota(jnp.int32, (NUM_RELS, BLK), 0)
               == e[None, :]).astype(f32)
    r = (jnp.dot(tt_ref[...], onehotT, preferred_element_type=f32)
         + rTb_ref[...].T)
    rn = jnp.sqrt(jnp.sum(r * r, axis=0, keepdims=True))
    r = r / jnp.maximum(rn, 1e-12)

    dp = h + r - p
    dn = h + r - t
    pos_ref[0, 0, :] = jnp.sqrt(jnp.sum(dp * dp, axis=0))
    neg_ref[0, 0, :] = jnp.sqrt(jnp.sum(dn * dn, axis=0))


def kernel(h_in_inc, h_out_inc, h_node_feat, eid, pos_t_in_inc, pos_t_out_inc,
           pos_t_node_feat, neg_t_in_inc, neg_t_out_inc, neg_t_node_feat,
           rel_emb, in_W, in_b, out_W, out_b, ent_W, ent_b, relT_W, relT_b):
    eid2 = eid.astype(jnp.int32).reshape(1, B)
    incT_spec = pl.BlockSpec((NUM_RELS, BLK), lambda i: (0, i))
    feat_spec = pl.BlockSpec((BLK, FEAT_DIM), lambda i: (i, 0))
    eid_spec = pl.BlockSpec((1, BLK), lambda i: (0, i))

    def full(shape):
        return pl.BlockSpec(shape, lambda i: (0,) * len(shape))

    out_spec = pl.BlockSpec((1, 1, BLK), lambda i: (i, 0, 0))
    pos, neg = pl.pallas_call(
        _fused_kernel,
        grid=(GRID,),
        in_specs=[incT_spec, incT_spec, feat_spec,
                  incT_spec, incT_spec, feat_spec,
                  incT_spec, incT_spec, feat_spec,
                  eid_spec,
                  full((NUM_RELS, REL_DIM)),
                  full((REL_DIM, REL_DIM)), full((1, REL_DIM)),
                  full((REL_DIM, REL_DIM)), full((1, REL_DIM)),
                  full((FEAT_DIM + 2 * REL_DIM, OUT_DIM)), full((1, OUT_DIM)),
                  full((REL_DIM, OUT_DIM)), full((1, OUT_DIM))],
        out_specs=[out_spec, out_spec],
        out_shape=[jax.ShapeDtypeStruct((GRID, 1, BLK), jnp.float32),
                   jax.ShapeDtypeStruct((GRID, 1, BLK), jnp.float32)],
        scratch_shapes=[pltpu.VMEM((REL_DIM, NUM_RELS), jnp.float32),
                        pltpu.VMEM((REL_DIM, NUM_RELS), jnp.float32),
                        pltpu.VMEM((OUT_DIM, NUM_RELS), jnp.float32),
                        pltpu.VMEM((1, OUT_DIM), jnp.float32)],
    )(h_in_inc.T, h_out_inc.T, h_node_feat,
      pos_t_in_inc.T, pos_t_out_inc.T, pos_t_node_feat,
      neg_t_in_inc.T, neg_t_out_inc.T, neg_t_node_feat,
      eid2, rel_emb,
      in_W, in_b.reshape(1, REL_DIM), out_W, out_b.reshape(1, REL_DIM),
      ent_W, ent_b.reshape(1, OUT_DIM), relT_W, relT_b.reshape(1, OUT_DIM))
    return pos.reshape(BPAD)[:B], neg.reshape(BPAD)[:B]


# speedup vs baseline: 1.3936x; 1.0128x over previous
import jax
import jax.numpy as jnp
from jax.experimental import pallas as pl
from jax.experimental.pallas import tpu as pltpu

B = 10000
NUM_RELS = 1315
FEAT_DIM = 256
REL_DIM = 32
OUT_DIM = 32
BLK = 512
GRID = -(-B // BLK)
BPAD = GRID * BLK


def _fused_kernel(hi_ref, ho_ref, hf_ref, pi_ref, po_ref, pf_ref,
                  ni_ref, no_ref, nf_ref, eid_ref, rel_ref,
                  inW_ref, inb_ref, outW_ref, outb_ref,
                  entW_ref, entb_ref, rTW_ref, rTb_ref,
                  pos_ref, neg_ref,
                  mtin_ref, mtout_ref, tt_ref, bias_ref):
    f32 = jnp.float32

    @pl.when(pl.program_id(0) == 0)
    def _prep():
        E1 = entW_ref[FEAT_DIM:FEAT_DIM + REL_DIM, :]
        E2 = entW_ref[FEAT_DIM + REL_DIM:, :]
        C_in = jnp.dot(inW_ref[...], E1, preferred_element_type=f32)
        C_out = jnp.dot(outW_ref[...], E2, preferred_element_type=f32)
        mtin_ref[...] = jnp.dot(rel_ref[...], C_in,
                                preferred_element_type=f32).T
        mtout_ref[...] = jnp.dot(rel_ref[...], C_out,
                                 preferred_element_type=f32).T
        tt_ref[...] = jnp.dot(rel_ref[...], rTW_ref[...],
                              preferred_element_type=f32).T
        bias_ref[...] = (jnp.dot(inb_ref[...], E1, preferred_element_type=f32)
                         + jnp.dot(outb_ref[...], E2,
                                   preferred_element_type=f32)
                         + entb_ref[...])

    E0 = entW_ref[:FEAT_DIM, :]
    biasT = bias_ref[...].T

    def embed(incT_in, incT_out, nf):
        f = jnp.dot(nf, E0, preferred_element_type=f32).T
        z = (f
             + jnp.dot(mtin_ref[...], incT_in, preferred_element_type=f32)
             + jnp.dot(mtout_ref[...], incT_out, preferred_element_type=f32)
             + biasT)
        n = jnp.sqrt(jnp.sum(z * z, axis=0, keepdims=True))
        return z / jnp.maximum(n, 1e-12)

    h = embed(hi_ref[...], ho_ref[...], hf_ref[...])
    p = embed(pi_ref[...], po_ref[...], pf_ref[...])
    t = embed(ni_ref[...], no_ref[...], nf_ref[...])

    e = eid_ref[0, :]
    onehotT = (jax.lax.broadcasted_iota(jnp.int32, (NUM_RELS, BLK), 0)
               == e[None, :]).astype(f32)
    r = (jnp.dot(tt_ref[...], onehotT, preferred_element_type=f32)
         + rTb_ref[...].T)
    rn = jnp.sqrt(jnp.sum(r * r, axis=0, keepdims=True))
    r = r / jnp.maximum(rn, 1e-12)

    dp = h + r - p
    dn = h + r - t
    pos_ref[0, 0, :] = jnp.sqrt(jnp.sum(dp * dp, axis=0))
    neg_ref[0, 0, :] = jnp.sqrt(jnp.sum(dn * dn, axis=0))


def kernel(h_in_inc, h_out_inc, h_node_feat, eid, pos_t_in_inc, pos_t_out_inc,
           pos_t_node_feat, neg_t_in_inc, neg_t_out_inc, neg_t_node_feat,
           rel_emb, in_W, in_b, out_W, out_b, ent_W, ent_b, relT_W, relT_b):
    eid2 = eid.astype(jnp.int32).reshape(1, B)
    incT_spec = pl.BlockSpec((NUM_RELS, BLK), lambda i: (0, i))
    feat_spec = pl.BlockSpec((BLK, FEAT_DIM), lambda i: (i, 0))
    eid_spec = pl.BlockSpec((1, BLK), lambda i: (0, i))

    def full(shape):
        return pl.BlockSpec(shape, lambda i: (0,) * len(shape))

    out_spec = pl.BlockSpec((1, 1, BLK), lambda i: (i, 0, 0))
    pos, neg = pl.pallas_call(
        _fused_kernel,
        grid=(GRID,),
        in_specs=[incT_spec, incT_spec, feat_spec,
                  incT_spec, incT_spec, feat_spec,
                  incT_spec, incT_spec, feat_spec,
                  eid_spec,
                  full((NUM_RELS, REL_DIM)),
                  full((REL_DIM, REL_DIM)), full((1, REL_DIM)),
                  full((REL_DIM, REL_DIM)), full((1, REL_DIM)),
                  full((FEAT_DIM + 2 * REL_DIM, OUT_DIM)), full((1, OUT_DIM)),
                  full((REL_DIM, OUT_DIM)), full((1, OUT_DIM))],
        out_specs=[out_spec, out_spec],
        out_shape=[jax.ShapeDtypeStruct((GRID, 1, BLK), jnp.float32),
                   jax.ShapeDtypeStruct((GRID, 1, BLK), jnp.float32)],
        scratch_shapes=[pltpu.VMEM((REL_DIM, NUM_RELS), jnp.float32),
                        pltpu.VMEM((REL_DIM, NUM_RELS), jnp.float32),
                        pltpu.VMEM((OUT_DIM, NUM_RELS), jnp.float32),
                        pltpu.VMEM((1, OUT_DIM), jnp.float32)],
    )(h_in_inc.T, h_out_inc.T, h_node_feat,
      pos_t_in_inc.T, pos_t_out_inc.T, pos_t_node_feat,
      neg_t_in_inc.T, neg_t_out_inc.T, neg_t_node_feat,
      eid2, rel_emb,
      in_W, in_b.reshape(1, REL_DIM), out_W, out_b.reshape(1, REL_DIM),
      ent_W, ent_b.reshape(1, OUT_DIM), relT_W, relT_b.reshape(1, OUT_DIM))
    return pos.reshape(BPAD)[:B], neg.reshape(BPAD)[:B]


# speedup vs baseline: 1.4686x; 1.0538x over previous
import jax
import jax.numpy as jnp
from jax.experimental import pallas as pl
from jax.experimental.pallas import tpu as pltpu

B = 10000
NUM_RELS = 1315
FEAT_DIM = 256
REL_DIM = 32
OUT_DIM = 32
BLK = 512
GRID = -(-B // BLK)
BPAD = GRID * BLK


def _fused_kernel(hi_ref, ho_ref, hf_ref, pi_ref, po_ref, pf_ref,
                  ni_ref, no_ref, nf_ref, eid_ref, rel_ref,
                  inW_ref, inb_ref, outW_ref, outb_ref,
                  entWT_ref, entb_ref, rTW_ref, rTb_ref,
                  pos_ref, neg_ref,
                  mtin_ref, mtout_ref, tt_ref, e0_ref, bias_ref):
    f32 = jnp.float32

    @pl.when(pl.program_id(0) == 0)
    def _prep():
        E1T = entWT_ref[:, FEAT_DIM:FEAT_DIM + REL_DIM]
        E2T = entWT_ref[:, FEAT_DIM + REL_DIM:]
        mtin_ref[...] = jnp.dot(
            jnp.dot(E1T, inW_ref[...].T, preferred_element_type=f32),
            rel_ref[...], preferred_element_type=f32)
        mtout_ref[...] = jnp.dot(
            jnp.dot(E2T, outW_ref[...].T, preferred_element_type=f32),
            rel_ref[...], preferred_element_type=f32)
        tt_ref[...] = jnp.dot(rTW_ref[...].T, rel_ref[...],
                              preferred_element_type=f32)
        e0_ref[...] = entWT_ref[:, :FEAT_DIM].T
        bias_ref[...] = (jnp.dot(inb_ref[...], E1T.T,
                                 preferred_element_type=f32)
                         + jnp.dot(outb_ref[...], E2T.T,
                                   preferred_element_type=f32)
                         + entb_ref[...])

    E0 = e0_ref[...]
    biasT = bias_ref[...].T

    def embed(incT_in, incT_out, nf):
        f = jnp.dot(nf, E0, preferred_element_type=f32).T
        z = (f
             + jnp.dot(mtin_ref[...], incT_in, preferred_element_type=f32)
             + jnp.dot(mtout_ref[...], incT_out, preferred_element_type=f32)
             + biasT)
        n = jnp.sqrt(jnp.sum(z * z, axis=0, keepdims=True))
        return z / jnp.maximum(n, 1e-12)

    h = embed(hi_ref[...], ho_ref[...], hf_ref[...])
    p = embed(pi_ref[...], po_ref[...], pf_ref[...])
    t = embed(ni_ref[...], no_ref[...], nf_ref[...])

    e = eid_ref[...]
    onehotT = (jax.lax.broadcasted_iota(jnp.int32, (NUM_RELS, BLK), 0)
               == e[None, :]).astype(f32)
    r = (jnp.dot(tt_ref[...], onehotT, preferred_element_type=f32)
         + rTb_ref[...].T)
    rn = jnp.sqrt(jnp.sum(r * r, axis=0, keepdims=True))
    r = r / jnp.maximum(rn, 1e-12)

    dp = h + r - p
    dn = h + r - t
    pos_ref[0, 0, :] = jnp.sqrt(jnp.sum(dp * dp, axis=0))
    neg_ref[0, 0, :] = jnp.sqrt(jnp.sum(dn * dn, axis=0))


def kernel(h_in_inc, h_out_inc, h_node_feat, eid, pos_t_in_inc, pos_t_out_inc,
           pos_t_node_feat, neg_t_in_inc, neg_t_out_inc, neg_t_node_feat,
           rel_emb, in_W, in_b, out_W, out_b, ent_W, ent_b, relT_W, relT_b):
    eid2 = eid.astype(jnp.int32)
    incT_spec = pl.BlockSpec((NUM_RELS, BLK), lambda i: (0, i))
    feat_spec = pl.BlockSpec((BLK, FEAT_DIM), lambda i: (i, 0))
    eid_spec = pl.BlockSpec((BLK,), lambda i: (i,))

    def full(shape):
        return pl.BlockSpec(shape, lambda i: (0,) * len(shape))

    out_spec = pl.BlockSpec((1, 1, BLK), lambda i: (i, 0, 0))
    pos, neg = pl.pallas_call(
        _fused_kernel,
        grid=(GRID,),
        in_specs=[incT_spec, incT_spec, feat_spec,
                  incT_spec, incT_spec, feat_spec,
                  incT_spec, incT_spec, feat_spec,
                  eid_spec,
                  full((REL_DIM, NUM_RELS)),
                  full((REL_DIM, REL_DIM)), full((1, REL_DIM)),
                  full((REL_DIM, REL_DIM)), full((1, REL_DIM)),
                  full((OUT_DIM, FEAT_DIM + 2 * REL_DIM)), full((1, OUT_DIM)),
                  full((REL_DIM, OUT_DIM)), full((1, OUT_DIM))],
        out_specs=[out_spec, out_spec],
        out_shape=[jax.ShapeDtypeStruct((GRID, 1, BLK), jnp.float32),
                   jax.ShapeDtypeStruct((GRID, 1, BLK), jnp.float32)],
        scratch_shapes=[pltpu.VMEM((REL_DIM, NUM_RELS), jnp.float32),
                        pltpu.VMEM((REL_DIM, NUM_RELS), jnp.float32),
                        pltpu.VMEM((OUT_DIM, NUM_RELS), jnp.float32),
                        pltpu.VMEM((FEAT_DIM, OUT_DIM), jnp.float32),
                        pltpu.VMEM((1, OUT_DIM), jnp.float32)],
    )(h_in_inc.T, h_out_inc.T, h_node_feat,
      pos_t_in_inc.T, pos_t_out_inc.T, pos_t_node_feat,
      neg_t_in_inc.T, neg_t_out_inc.T, neg_t_node_feat,
      eid2, rel_emb.T,
      in_W, in_b.reshape(1, REL_DIM), out_W, out_b.reshape(1, REL_DIM),
      ent_W.T, ent_b.reshape(1, OUT_DIM), relT_W, relT_b.reshape(1, OUT_DIM))
    return pos.reshape(BPAD)[:B], neg.reshape(BPAD)[:B]


# speedup vs baseline: 1.4861x; 1.0119x over previous
import jax
import jax.numpy as jnp
from jax.experimental import pallas as pl
from jax.experimental.pallas import tpu as pltpu

B = 10000
NUM_RELS = 1315
FEAT_DIM = 256
REL_DIM = 32
OUT_DIM = 32
BLK = 256
GRID = -(-B // BLK)
BPAD = GRID * BLK


def _fused_kernel(hi_ref, ho_ref, hf_ref, pi_ref, po_ref, pf_ref,
                  ni_ref, no_ref, nf_ref, eid_ref, rel_ref,
                  inW_ref, inb_ref, outW_ref, outb_ref,
                  entWT_ref, entb_ref, rTW_ref, rTb_ref,
                  pos_ref, neg_ref,
                  mtin_ref, mtout_ref, tt_ref, e0_ref, bias_ref):
    f32 = jnp.float32

    @pl.when(pl.program_id(0) == 0)
    def _prep():
        E1T = entWT_ref[:, FEAT_DIM:FEAT_DIM + REL_DIM]
        E2T = entWT_ref[:, FEAT_DIM + REL_DIM:]
        mtin_ref[...] = jnp.dot(
            jnp.dot(E1T, inW_ref[...].T, preferred_element_type=f32),
            rel_ref[...], preferred_element_type=f32)
        mtout_ref[...] = jnp.dot(
            jnp.dot(E2T, outW_ref[...].T, preferred_element_type=f32),
            rel_ref[...], preferred_element_type=f32)
        tt_ref[...] = jnp.dot(rTW_ref[...].T, rel_ref[...],
                              preferred_element_type=f32)
        e0_ref[...] = entWT_ref[:, :FEAT_DIM].T
        bias_ref[...] = (jnp.dot(inb_ref[...], E1T.T,
                                 preferred_element_type=f32)
                         + jnp.dot(outb_ref[...], E2T.T,
                                   preferred_element_type=f32)
                         + entb_ref[...])

    E0 = e0_ref[...]
    biasT = bias_ref[...].T

    def embed(incT_in, incT_out, nf):
        f = jnp.dot(nf, E0, preferred_element_type=f32).T
        z = (f
             + jnp.dot(mtin_ref[...], incT_in, preferred_element_type=f32)
             + jnp.dot(mtout_ref[...], incT_out, preferred_element_type=f32)
             + biasT)
        n = jnp.sqrt(jnp.sum(z * z, axis=0, keepdims=True))
        return z / jnp.maximum(n, 1e-12)

    h = embed(hi_ref[...], ho_ref[...], hf_ref[...])
    p = embed(pi_ref[...], po_ref[...], pf_ref[...])
    t = embed(ni_ref[...], no_ref[...], nf_ref[...])

    e = eid_ref[...]
    onehotT = (jax.lax.broadcasted_iota(jnp.int32, (NUM_RELS, BLK), 0)
               == e[None, :]).astype(f32)
    r = (jnp.dot(tt_ref[...], onehotT, preferred_element_type=f32)
         + rTb_ref[...].T)
    rn = jnp.sqrt(jnp.sum(r * r, axis=0, keepdims=True))
    r = r / jnp.maximum(rn, 1e-12)

    dp = h + r - p
    dn = h + r - t
    pos_ref[0, 0, :] = jnp.sqrt(jnp.sum(dp * dp, axis=0))
    neg_ref[0, 0, :] = jnp.sqrt(jnp.sum(dn * dn, axis=0))


def kernel(h_in_inc, h_out_inc, h_node_feat, eid, pos_t_in_inc, pos_t_out_inc,
           pos_t_node_feat, neg_t_in_inc, neg_t_out_inc, neg_t_node_feat,
           rel_emb, in_W, in_b, out_W, out_b, ent_W, ent_b, relT_W, relT_b):
    eid2 = eid.astype(jnp.int32)
    incT_spec = pl.BlockSpec((NUM_RELS, BLK), lambda i: (0, i))
    feat_spec = pl.BlockSpec((BLK, FEAT_DIM), lambda i: (i, 0))
    eid_spec = pl.BlockSpec((BLK,), lambda i: (i,))

    def full(shape):
        return pl.BlockSpec(shape, lambda i: (0,) * len(shape))

    out_spec = pl.BlockSpec((1, 1, BLK), lambda i: (i, 0, 0))
    pos, neg = pl.pallas_call(
        _fused_kernel,
        grid=(GRID,),
        in_specs=[incT_spec, incT_spec, feat_spec,
                  incT_spec, incT_spec, feat_spec,
                  incT_spec, incT_spec, feat_spec,
                  eid_spec,
                  full((REL_DIM, NUM_RELS)),
                  full((REL_DIM, REL_DIM)), full((1, REL_DIM)),
                  full((REL_DIM, REL_DIM)), full((1, REL_DIM)),
                  full((OUT_DIM, FEAT_DIM + 2 * REL_DIM)), full((1, OUT_DIM)),
                  full((REL_DIM, OUT_DIM)), full((1, OUT_DIM))],
        out_specs=[out_spec, out_spec],
        out_shape=[jax.ShapeDtypeStruct((GRID, 1, BLK), jnp.float32),
                   jax.ShapeDtypeStruct((GRID, 1, BLK), jnp.float32)],
        scratch_shapes=[pltpu.VMEM((REL_DIM, NUM_RELS), jnp.float32),
                        pltpu.VMEM((REL_DIM, NUM_RELS), jnp.float32),
                        pltpu.VMEM((OUT_DIM, NUM_RELS), jnp.float32),
                        pltpu.VMEM((FEAT_DIM, OUT_DIM), jnp.float32),
                        pltpu.VMEM((1, OUT_DIM), jnp.float32)],
    )(h_in_inc.T, h_out_inc.T, h_node_feat,
      pos_t_in_inc.T, pos_t_out_inc.T, pos_t_node_feat,
      neg_t_in_inc.T, neg_t_out_inc.T, neg_t_node_feat,
      eid2, rel_emb.T,
      in_W, in_b.reshape(1, REL_DIM), out_W, out_b.reshape(1, REL_DIM),
      ent_W.T, ent_b.reshape(1, OUT_DIM), relT_W, relT_b.reshape(1, OUT_DIM))
    return pos.reshape(BPAD)[:B], neg.reshape(BPAD)[:B]
